# baseline (speedup 1.0000x reference)
import jax
import jax.numpy as jnp
from jax.experimental import pallas as pl
from jax.experimental.pallas import tpu as pltpu

BLK = 128
NB = 16
KEEP = 2
NEG = -1e9
FMIN = -3.0e38


def _mask_kernel(q_ref, k_ref, idx_ref):
    q = q_ref[0, 0]
    k = k_ref[0, 0]
    S, D = q.shape
    scale = jnp.float32(1.0) / jnp.sqrt(jnp.float32(D))
    qp = jnp.concatenate(
        [jnp.sum(q[i * BLK:(i + 1) * BLK, :], axis=0, keepdims=True)
         for i in range(NB)], axis=0) * jnp.float32(1.0 / BLK)
    kp = jnp.concatenate(
        [jnp.sum(k[i * BLK:(i + 1) * BLK, :], axis=0, keepdims=True)
         for i in range(NB)], axis=0) * jnp.float32(1.0 / BLK)
    s = jax.lax.dot_general(qp.astype(jnp.bfloat16), kp.astype(jnp.bfloat16),
                            (((1,), (1,)), ((), ())),
                            preferred_element_type=jnp.float32) * scale
    m = jnp.max(s, axis=1, keepdims=True)
    e = jnp.exp(s - m)
    p = e / jnp.sum(e, axis=1, keepdims=True)
    col = jax.lax.broadcasted_iota(jnp.int32, (NB, NB), 1)
    m1 = jnp.max(p, axis=1, keepdims=True)
    a1 = jnp.min(jnp.where(p >= m1, col, NB), axis=1)
    p2 = jnp.where(col == a1[:, None], FMIN, p)
    m2 = jnp.max(p2, axis=1, keepdims=True)
    a2 = jnp.min(jnp.where(p2 >= m2, col, NB), axis=1)
    idx_ref[0] = jnp.stack([a1, a2], axis=0)


def _attn_kernel(idx_ref, q_ref, k_ref, v_ref, o_ref):
    b = pl.program_id(0)
    h = pl.program_id(1)
    row = b * pl.num_programs(1) + h
    scale = jnp.float32(0.125)
    for qb in range(NB):
        i0 = idx_ref[row, 0, qb]
        i1 = idx_ref[row, 1, qb]
        q = q_ref[0, 0, qb * BLK:(qb + 1) * BLK, :].astype(jnp.bfloat16)
        k0 = k_ref[0, 0, pl.ds(i0 * BLK, BLK), :].astype(jnp.bfloat16)
        k1 = k_ref[0, 0, pl.ds(i1 * BLK, BLK), :].astype(jnp.bfloat16)
        kd = k_ref[0, 0, qb * BLK:(qb + 1) * BLK, :].astype(jnp.bfloat16)
        s0 = jnp.dot(q, k0.T, preferred_element_type=jnp.float32) * scale
        s1 = jnp.dot(q, k1.T, preferred_element_type=jnp.float32) * scale
        sd = jnp.dot(q, kd.T, preferred_element_type=jnp.float32) * scale
        dup = jnp.logical_or(i0 == qb, i1 == qb)
        sd = jnp.where(dup, NEG, sd)
        m = jnp.maximum(jnp.maximum(jnp.max(s0, axis=1), jnp.max(s1, axis=1)),
                        jnp.max(sd, axis=1))[:, None]
        p0 = jnp.exp(s0 - m)
        p1 = jnp.exp(s1 - m)
        pd = jnp.exp(sd - m)
        denom = (jnp.sum(p0, axis=1) + jnp.sum(p1, axis=1)
                 + jnp.sum(pd, axis=1))[:, None]
        p0 = (p0 / denom).astype(jnp.bfloat16)
        p1 = (p1 / denom).astype(jnp.bfloat16)
        pd = (pd / denom).astype(jnp.bfloat16)
        v0 = v_ref[0, 0, pl.ds(i0 * BLK, BLK), :].astype(jnp.bfloat16)
        v1 = v_ref[0, 0, pl.ds(i1 * BLK, BLK), :].astype(jnp.bfloat16)
        vd = v_ref[0, 0, qb * BLK:(qb + 1) * BLK, :].astype(jnp.bfloat16)
        acc = jnp.dot(p0, v0, preferred_element_type=jnp.float32)
        acc = acc + jnp.dot(p1, v1, preferred_element_type=jnp.float32)
        acc = acc + jnp.dot(pd, vd, preferred_element_type=jnp.float32)
        o_ref[0, 0, qb * BLK:(qb + 1) * BLK, :] = acc


def kernel(q, k, v):
    B, H, S, D = q.shape

    idx = pl.pallas_call(
        _mask_kernel,
        grid=(B, H),
        in_specs=[
            pl.BlockSpec((1, 1, S, D), lambda b, h: (b, h, 0, 0)),
            pl.BlockSpec((1, 1, S, D), lambda b, h: (b, h, 0, 0)),
        ],
        out_specs=pl.BlockSpec((1, 2, NB), lambda b, h: (b * H + h, 0, 0)),
        out_shape=jax.ShapeDtypeStruct((B * H, 2, NB), jnp.int32),
    )(q, k)

    out = pl.pallas_call(
        _attn_kernel,
        grid_spec=pltpu.PrefetchScalarGridSpec(
            num_scalar_prefetch=1,
            grid=(B, H),
            in_specs=[
                pl.BlockSpec((1, 1, S, D), lambda b, h, idx_ref: (b, h, 0, 0)),
                pl.BlockSpec((1, 1, S, D), lambda b, h, idx_ref: (b, h, 0, 0)),
                pl.BlockSpec((1, 1, S, D), lambda b, h, idx_ref: (b, h, 0, 0)),
            ],
            out_specs=pl.BlockSpec((1, 1, S, D),
                                   lambda b, h, idx_ref: (b, h, 0, 0)),
        ),
        out_shape=jax.ShapeDtypeStruct((B, H, S, D), jnp.float32),
    )(idx, q, k, v)

    return out


# speedup vs baseline: 1.1906x; 1.1906x over previous
import jax
import jax.numpy as jnp
from jax.experimental import pallas as pl
from jax.experimental.pallas import tpu as pltpu

BLK = 128
NB = 16
KEEP = 2
NEG = -1e9
FMIN = -3.0e38


def _mask_kernel(q_ref, k_ref, idx_ref):
    q = q_ref[0, 0]
    k = k_ref[0, 0]
    S, D = q.shape
    scale = jnp.float32(1.0) / jnp.sqrt(jnp.float32(D))
    qp = jnp.concatenate(
        [jnp.sum(q[i * BLK:(i + 1) * BLK, :], axis=0, keepdims=True)
         for i in range(NB)], axis=0) * jnp.float32(1.0 / BLK)
    kp = jnp.concatenate(
        [jnp.sum(k[i * BLK:(i + 1) * BLK, :], axis=0, keepdims=True)
         for i in range(NB)], axis=0) * jnp.float32(1.0 / BLK)
    s = jax.lax.dot_general(qp.astype(jnp.bfloat16), kp.astype(jnp.bfloat16),
                            (((1,), (1,)), ((), ())),
                            preferred_element_type=jnp.float32) * scale
    m = jnp.max(s, axis=1, keepdims=True)
    e = jnp.exp(s - m)
    p = e / jnp.sum(e, axis=1, keepdims=True)
    col = jax.lax.broadcasted_iota(jnp.int32, (NB, NB), 1)
    m1 = jnp.max(p, axis=1, keepdims=True)
    a1 = jnp.min(jnp.where(p >= m1, col, NB), axis=1)
    p2 = jnp.where(col == a1[:, None], FMIN, p)
    m2 = jnp.max(p2, axis=1, keepdims=True)
    a2 = jnp.min(jnp.where(p2 >= m2, col, NB), axis=1)
    idx_ref[0] = jnp.stack([a1, a2], axis=0)


def _attn_kernel(idx_ref, q_ref, k_ref, v_ref, o_ref):
    b = pl.program_id(0)
    h = pl.program_id(1)
    row = b * pl.num_programs(1) + h
    scale = jnp.float32(0.125)
    for qb in range(NB):
        i0 = idx_ref[row, 0, qb]
        i1 = idx_ref[row, 1, qb]
        q = q_ref[0, 0, qb * BLK:(qb + 1) * BLK, :]
        k0 = k_ref[0, 0, pl.ds(i0 * BLK, BLK), :]
        k1 = k_ref[0, 0, pl.ds(i1 * BLK, BLK), :]
        kd = k_ref[0, 0, qb * BLK:(qb + 1) * BLK, :]
        s0 = jnp.dot(q, k0.T, preferred_element_type=jnp.float32) * scale
        s1 = jnp.dot(q, k1.T, preferred_element_type=jnp.float32) * scale
        sd = jnp.dot(q, kd.T, preferred_element_type=jnp.float32) * scale
        dup = jnp.logical_or(i0 == qb, i1 == qb)
        sd = jnp.where(dup, NEG, sd)
        m = jnp.maximum(jnp.maximum(jnp.max(s0, axis=1), jnp.max(s1, axis=1)),
                        jnp.max(sd, axis=1))[:, None]
        p0 = jnp.exp(s0 - m)
        p1 = jnp.exp(s1 - m)
        pd = jnp.exp(sd - m)
        denom = (jnp.sum(p0, axis=1) + jnp.sum(p1, axis=1)
                 + jnp.sum(pd, axis=1))[:, None]
        v0 = v_ref[0, 0, pl.ds(i0 * BLK, BLK), :]
        v1 = v_ref[0, 0, pl.ds(i1 * BLK, BLK), :]
        vd = v_ref[0, 0, qb * BLK:(qb + 1) * BLK, :]
        acc = jnp.dot(p0, v0, preferred_element_type=jnp.float32)
        acc = acc + jnp.dot(p1, v1, preferred_element_type=jnp.float32)
        acc = acc + jnp.dot(pd, vd, preferred_element_type=jnp.float32)
        o_ref[0, 0, qb * BLK:(qb + 1) * BLK, :] = acc / denom


def kernel(q, k, v):
    B, H, S, D = q.shape

    idx = pl.pallas_call(
        _mask_kernel,
        grid=(B, H),
        in_specs=[
            pl.BlockSpec((1, 1, S, D), lambda b, h: (b, h, 0, 0)),
            pl.BlockSpec((1, 1, S, D), lambda b, h: (b, h, 0, 0)),
        ],
        out_specs=pl.BlockSpec((1, 2, NB), lambda b, h: (b * H + h, 0, 0)),
        out_shape=jax.ShapeDtypeStruct((B * H, 2, NB), jnp.int32),
    )(q, k)

    out = pl.pallas_call(
        _attn_kernel,
        grid_spec=pltpu.PrefetchScalarGridSpec(
            num_scalar_prefetch=1,
            grid=(B, H),
            in_specs=[
                pl.BlockSpec((1, 1, S, D), lambda b, h, idx_ref: (b, h, 0, 0)),
                pl.BlockSpec((1, 1, S, D), lambda b, h, idx_ref: (b, h, 0, 0)),
                pl.BlockSpec((1, 1, S, D), lambda b, h, idx_ref: (b, h, 0, 0)),
            ],
            out_specs=pl.BlockSpec((1, 1, S, D),
                                   lambda b, h, idx_ref: (b, h, 0, 0)),
        ),
        out_shape=jax.ShapeDtypeStruct((B, H, S, D), jnp.float32),
    )(idx, q, k, v)

    return out


# speedup vs baseline: 1.2605x; 1.0588x over previous
import jax
import jax.numpy as jnp
from jax.experimental import pallas as pl
from jax.experimental.pallas import tpu as pltpu

BLK = 128
NB = 16
KEEP = 2
NEG = -1e9
FMIN = -3.0e38


def _mask_kernel(q_ref, k_ref, idx_ref):
    q = q_ref[0, 0]
    k = k_ref[0, 0]
    S, D = q.shape
    scale = jnp.float32(1.0) / jnp.sqrt(jnp.float32(D))
    qp = jnp.concatenate(
        [jnp.sum(q[i * BLK:(i + 1) * BLK, :], axis=0, keepdims=True)
         for i in range(NB)], axis=0) * jnp.float32(1.0 / BLK)
    kp = jnp.concatenate(
        [jnp.sum(k[i * BLK:(i + 1) * BLK, :], axis=0, keepdims=True)
         for i in range(NB)], axis=0) * jnp.float32(1.0 / BLK)
    s = jax.lax.dot_general(qp.astype(jnp.bfloat16), kp.astype(jnp.bfloat16),
                            (((1,), (1,)), ((), ())),
                            preferred_element_type=jnp.float32) * scale
    m = jnp.max(s, axis=1, keepdims=True)
    e = jnp.exp(s - m)
    p = e / jnp.sum(e, axis=1, keepdims=True)
    col = jax.lax.broadcasted_iota(jnp.int32, (NB, NB), 1)
    m1 = jnp.max(p, axis=1, keepdims=True)
    a1 = jnp.min(jnp.where(p >= m1, col, NB), axis=1)
    p2 = jnp.where(col == a1[:, None], FMIN, p)
    m2 = jnp.max(p2, axis=1, keepdims=True)
    a2 = jnp.min(jnp.where(p2 >= m2, col, NB), axis=1)
    idx_ref[0] = jnp.stack([a1, a2], axis=0)


def _attn_kernel(idx_ref, q_ref, k_ref, v_ref, o_ref):
    b = pl.program_id(0)
    h = pl.program_id(1)
    row = b * pl.num_programs(1) + h
    scale = jnp.float32(0.125)
    for qb in range(NB):
        i0 = idx_ref[row, 0, qb]
        i1 = idx_ref[row, 1, qb]
        q = q_ref[0, 0, qb * BLK:(qb + 1) * BLK, :]
        kc = jnp.concatenate(
            [k_ref[0, 0, pl.ds(i0 * BLK, BLK), :],
             k_ref[0, 0, pl.ds(i1 * BLK, BLK), :],
             k_ref[0, 0, qb * BLK:(qb + 1) * BLK, :]], axis=0)
        vc = jnp.concatenate(
            [v_ref[0, 0, pl.ds(i0 * BLK, BLK), :],
             v_ref[0, 0, pl.ds(i1 * BLK, BLK), :],
             v_ref[0, 0, qb * BLK:(qb + 1) * BLK, :]], axis=0)
        s = jnp.dot(q, kc.T, preferred_element_type=jnp.float32) * scale
        dup = jnp.logical_or(i0 == qb, i1 == qb)
        colmask = jax.lax.broadcasted_iota(jnp.int32, (1, 3 * BLK), 1) >= 2 * BLK
        s = jnp.where(jnp.logical_and(dup, colmask), NEG, s)
        m = jnp.max(s, axis=1, keepdims=True)
        p = jnp.exp(s - m)
        denom = jnp.sum(p, axis=1, keepdims=True)
        acc = jnp.dot(p, vc, preferred_element_type=jnp.float32)
        o_ref[0, 0, qb * BLK:(qb + 1) * BLK, :] = acc / denom


def kernel(q, k, v):
    B, H, S, D = q.shape

    idx = pl.pallas_call(
        _mask_kernel,
        grid=(B, H),
        in_specs=[
            pl.BlockSpec((1, 1, S, D), lambda b, h: (b, h, 0, 0)),
            pl.BlockSpec((1, 1, S, D), lambda b, h: (b, h, 0, 0)),
        ],
        out_specs=pl.BlockSpec((1, 2, NB), lambda b, h: (b * H + h, 0, 0)),
        out_shape=jax.ShapeDtypeStruct((B * H, 2, NB), jnp.int32),
    )(q, k)

    out = pl.pallas_call(
        _attn_kernel,
        grid_spec=pltpu.PrefetchScalarGridSpec(
            num_scalar_prefetch=1,
            grid=(B, H),
            in_specs=[
                pl.BlockSpec((1, 1, S, D), lambda b, h, idx_ref: (b, h, 0, 0)),
                pl.BlockSpec((1, 1, S, D), lambda b, h, idx_ref: (b, h, 0, 0)),
                pl.BlockSpec((1, 1, S, D), lambda b, h, idx_ref: (b, h, 0, 0)),
            ],
            out_specs=pl.BlockSpec((1, 1, S, D),
                                   lambda b, h, idx_ref: (b, h, 0, 0)),
        ),
        out_shape=jax.ShapeDtypeStruct((B, H, S, D), jnp.float32),
    )(idx, q, k, v)

    return out
